# baseline (speedup 1.0000x reference)
import jax
import jax.numpy as jnp
from jax.experimental import pallas as pl

BLK = 16384


def _tc_body(w_ref, x_ref, o_ref):
    o_ref[...] = jnp.sum(x_ref[...] * w_ref[...], axis=0)


def kernel(items_emb, user_w):
    m, dim = items_emb.shape
    xt = items_emb.T
    w_col = user_w.reshape(dim, 1)
    grid = (m + BLK - 1) // BLK
    out = pl.pallas_call(
        _tc_body,
        grid=(grid,),
        in_specs=[
            pl.BlockSpec((dim, 1), lambda i: (0, 0)),
            pl.BlockSpec((dim, BLK), lambda i: (0, i)),
        ],
        out_specs=pl.BlockSpec((BLK,), lambda i: (i,)),
        out_shape=jax.ShapeDtypeStruct((m,), jnp.float32),
    )(w_col, xt)
    return out


# speedup vs baseline: 1.1713x; 1.1713x over previous
import jax
import jax.numpy as jnp
from jax.experimental import pallas as pl

BLK = 40960


def _tc_body(w_ref, x_ref, o_ref):
    o_ref[...] = jnp.sum(x_ref[...] * w_ref[...], axis=0)


def kernel(items_emb, user_w):
    m, dim = items_emb.shape
    xt = items_emb.T
    w_col = user_w.reshape(dim, 1)
    grid = (m + BLK - 1) // BLK
    out = pl.pallas_call(
        _tc_body,
        grid=(grid,),
        in_specs=[
            pl.BlockSpec((dim, 1), lambda i: (0, 0)),
            pl.BlockSpec((dim, BLK), lambda i: (0, i)),
        ],
        out_specs=pl.BlockSpec((BLK,), lambda i: (i,)),
        out_shape=jax.ShapeDtypeStruct((m,), jnp.float32),
    )(w_col, xt)
    return out
